# baseline (speedup 1.0000x reference)
import jax
import jax.numpy as jnp
from jax.experimental import pallas as pl
from jax.experimental.pallas import tpu as pltpu

_CH = 256
_NBUF = 16


def _make_tc_copy(batch, seq_len, d_model):
    nch = seq_len // _CH

    def body(emb_hbm, out_hbm, *rest):
        bufs = rest[:_NBUF]
        insem, outsem = rest[_NBUF], rest[_NBUF + 1]
        in_h = [None] * nch
        out_h = [None] * nch
        in_h[0] = pltpu.make_async_copy(emb_hbm.at[pl.ds(0, _CH)], bufs[0], insem)
        in_h[0].start()
        for c in range(nch):
            if c + 1 < nch:
                if c + 1 - _NBUF >= 0:
                    for h in out_h[c + 1 - _NBUF]:
                        h.wait()
                in_h[c + 1] = pltpu.make_async_copy(
                    emb_hbm.at[pl.ds((c + 1) * _CH, _CH)],
                    bufs[(c + 1) % _NBUF],
                    insem,
                )
                in_h[c + 1].start()
            in_h[c].wait()
            buf = bufs[c % _NBUF]
            out_h[c] = []
            for b in range(batch):
                h = pltpu.make_async_copy(
                    buf, out_hbm.at[pl.ds(b * seq_len + c * _CH, _CH)], outsem
                )
                h.start()
                out_h[c].append(h)
        for c in range(max(0, nch - _NBUF), nch):
            for h in out_h[c]:
                h.wait()

    return pl.pallas_call(
        body,
        in_specs=[pl.BlockSpec(memory_space=pl.ANY)],
        out_specs=pl.BlockSpec(memory_space=pl.ANY),
        out_shape=jax.ShapeDtypeStruct((batch * seq_len, d_model), jnp.float32),
        scratch_shapes=[pltpu.VMEM((_CH, d_model), jnp.float32) for _ in range(_NBUF)]
        + [pltpu.SemaphoreType.DMA, pltpu.SemaphoreType.DMA],
    )


def kernel(x, pos_embedding):
    batch, seq_len = x.shape
    max_len, d_model = pos_embedding.shape
    out_flat = _make_tc_copy(batch, seq_len, d_model)(pos_embedding)
    return out_flat.reshape(batch, seq_len, d_model)


# speedup vs baseline: 1.0700x; 1.0700x over previous
import jax
import jax.numpy as jnp
from jax.experimental import pallas as pl
from jax.experimental.pallas import tpu as pltpu

_CH = 1024
_NBUF = 4


def _make_tc_copy(batch, seq_len, d_model):
    nch = seq_len // _CH

    def body(emb_hbm, out_hbm, *rest):
        bufs = rest[:_NBUF]
        insem, outsem = rest[_NBUF], rest[_NBUF + 1]
        in_h = [None] * nch
        out_h = [None] * nch
        in_h[0] = pltpu.make_async_copy(emb_hbm.at[pl.ds(0, _CH)], bufs[0], insem)
        in_h[0].start()
        for c in range(nch):
            if c + 1 < nch:
                if c + 1 - _NBUF >= 0:
                    for h in out_h[c + 1 - _NBUF]:
                        h.wait()
                in_h[c + 1] = pltpu.make_async_copy(
                    emb_hbm.at[pl.ds((c + 1) * _CH, _CH)],
                    bufs[(c + 1) % _NBUF],
                    insem,
                )
                in_h[c + 1].start()
            in_h[c].wait()
            buf = bufs[c % _NBUF]
            out_h[c] = []
            for b in range(batch):
                h = pltpu.make_async_copy(
                    buf, out_hbm.at[pl.ds(b * seq_len + c * _CH, _CH)], outsem
                )
                h.start()
                out_h[c].append(h)
        for c in range(max(0, nch - _NBUF), nch):
            for h in out_h[c]:
                h.wait()

    return pl.pallas_call(
        body,
        in_specs=[pl.BlockSpec(memory_space=pl.ANY)],
        out_specs=pl.BlockSpec(memory_space=pl.ANY),
        out_shape=jax.ShapeDtypeStruct((batch * seq_len, d_model), jnp.float32),
        scratch_shapes=[pltpu.VMEM((_CH, d_model), jnp.float32) for _ in range(_NBUF)]
        + [pltpu.SemaphoreType.DMA, pltpu.SemaphoreType.DMA],
    )


def kernel(x, pos_embedding):
    batch, seq_len = x.shape
    max_len, d_model = pos_embedding.shape
    out_flat = _make_tc_copy(batch, seq_len, d_model)(pos_embedding)
    return out_flat.reshape(batch, seq_len, d_model)


# speedup vs baseline: 1.0891x; 1.0179x over previous
import jax
import jax.numpy as jnp
from jax.experimental import pallas as pl
from jax.experimental.pallas import tpu as pltpu

_CH = 2048
_NBUF = 4


def _make_tc_copy(batch, seq_len, d_model):
    nch = seq_len // _CH

    def body(emb_hbm, out_hbm, *rest):
        bufs = rest[:_NBUF]
        insem, outsem = rest[_NBUF], rest[_NBUF + 1]
        in_h = [None] * nch
        out_h = [None] * nch
        in_h[0] = pltpu.make_async_copy(emb_hbm.at[pl.ds(0, _CH)], bufs[0], insem)
        in_h[0].start()
        for c in range(nch):
            if c + 1 < nch:
                if c + 1 - _NBUF >= 0:
                    for h in out_h[c + 1 - _NBUF]:
                        h.wait()
                in_h[c + 1] = pltpu.make_async_copy(
                    emb_hbm.at[pl.ds((c + 1) * _CH, _CH)],
                    bufs[(c + 1) % _NBUF],
                    insem,
                )
                in_h[c + 1].start()
            in_h[c].wait()
            buf = bufs[c % _NBUF]
            out_h[c] = []
            for b in range(batch):
                h = pltpu.make_async_copy(
                    buf, out_hbm.at[pl.ds(b * seq_len + c * _CH, _CH)], outsem
                )
                h.start()
                out_h[c].append(h)
        for c in range(max(0, nch - _NBUF), nch):
            for h in out_h[c]:
                h.wait()

    return pl.pallas_call(
        body,
        in_specs=[pl.BlockSpec(memory_space=pl.ANY)],
        out_specs=pl.BlockSpec(memory_space=pl.ANY),
        out_shape=jax.ShapeDtypeStruct((batch * seq_len, d_model), jnp.float32),
        scratch_shapes=[pltpu.VMEM((_CH, d_model), jnp.float32) for _ in range(_NBUF)]
        + [pltpu.SemaphoreType.DMA, pltpu.SemaphoreType.DMA],
    )


def kernel(x, pos_embedding):
    batch, seq_len = x.shape
    max_len, d_model = pos_embedding.shape
    out_flat = _make_tc_copy(batch, seq_len, d_model)(pos_embedding)
    return out_flat.reshape(batch, seq_len, d_model)


# speedup vs baseline: 1.0914x; 1.0022x over previous
import jax
import jax.numpy as jnp
from jax.experimental import pallas as pl
from jax.experimental.pallas import tpu as pltpu

_CH = 4096
_NBUF = 2


def _make_tc_copy(batch, seq_len, d_model):
    nch = seq_len // _CH

    def body(emb_hbm, out_hbm, *rest):
        bufs = rest[:_NBUF]
        insem, outsem = rest[_NBUF], rest[_NBUF + 1]
        in_h = [None] * nch
        out_h = [None] * nch
        in_h[0] = pltpu.make_async_copy(emb_hbm.at[pl.ds(0, _CH)], bufs[0], insem)
        in_h[0].start()
        for c in range(nch):
            if c + 1 < nch:
                if c + 1 - _NBUF >= 0:
                    for h in out_h[c + 1 - _NBUF]:
                        h.wait()
                in_h[c + 1] = pltpu.make_async_copy(
                    emb_hbm.at[pl.ds((c + 1) * _CH, _CH)],
                    bufs[(c + 1) % _NBUF],
                    insem,
                )
                in_h[c + 1].start()
            in_h[c].wait()
            buf = bufs[c % _NBUF]
            out_h[c] = []
            for b in range(batch):
                h = pltpu.make_async_copy(
                    buf, out_hbm.at[pl.ds(b * seq_len + c * _CH, _CH)], outsem
                )
                h.start()
                out_h[c].append(h)
        for c in range(max(0, nch - _NBUF), nch):
            for h in out_h[c]:
                h.wait()

    return pl.pallas_call(
        body,
        in_specs=[pl.BlockSpec(memory_space=pl.ANY)],
        out_specs=pl.BlockSpec(memory_space=pl.ANY),
        out_shape=jax.ShapeDtypeStruct((batch * seq_len, d_model), jnp.float32),
        scratch_shapes=[pltpu.VMEM((_CH, d_model), jnp.float32) for _ in range(_NBUF)]
        + [pltpu.SemaphoreType.DMA, pltpu.SemaphoreType.DMA],
    )


def kernel(x, pos_embedding):
    batch, seq_len = x.shape
    max_len, d_model = pos_embedding.shape
    out_flat = _make_tc_copy(batch, seq_len, d_model)(pos_embedding)
    return out_flat.reshape(batch, seq_len, d_model)
